# baseline (speedup 1.0000x reference)
import jax
import jax.numpy as jnp
from jax import lax
from jax.experimental import pallas as pl
from jax.experimental.pallas import tpu as pltpu
from jax.experimental.pallas import tpu_sc as plsc

_NUM_NODES = 2000000
_LR_SCHEDULE = (0.1, 0.095, 0.09, 0.085, 0.08, 0.075, 0.07, 0.065, 0.06, 0.055)
_B = 1048576
_NC = 2
_NS = 16
_NW = _NC * _NS
_C = 4096
_N_W = _B // _NW
_CHUNKS = _N_W // _C


def _stress_body(i_hbm, j_hbm, vi_hbm, vj_hbm, dis_hbm, lr_hbm, pos_hbm,
                 out_hbm,
                 iv0, jv0, viv0, vjv0, disv0,
                 iv1, jv1, viv1, vjv1, disv1,
                 idx_xi, idx_yi, idx_xj, idx_yj,
                 xi0, yi0, xj0, yj0,
                 xi1, yi1, xj1, yj1,
                 lrv, accv, sem_in, sem_g):
  wid = lax.axis_index("s") * _NC + lax.axis_index("c")
  ins = ((iv0, jv0, viv0, vjv0, disv0), (iv1, jv1, viv1, vjv1, disv1))
  xys = ((xi0, yi0, xj0, yj0), (xi1, yi1, xj1, yj1))
  idxs = (idx_xi, idx_yi, idx_xj, idx_yj)
  pltpu.sync_copy(lr_hbm, lrv)
  accv[...] = jnp.zeros((16,), jnp.float32)
  lrvec = lrv[...]

  def issue_inputs(c, s):
    base = wid * _N_W + c * _C
    for src, dst in zip((i_hbm, j_hbm, vi_hbm, vj_hbm, dis_hbm), ins[s]):
      pltpu.async_copy(src.at[pl.ds(base, _C)], dst, sem_in)

  def drain_inputs(s):
    for src, dst in zip((i_hbm, j_hbm, vi_hbm, vj_hbm, dis_hbm), ins[s]):
      pltpu.make_async_copy(src.at[pl.ds(0, _C)], dst, sem_in).wait()

  def idx_compute(s):
    iv, jv, viv, vjv, _ = ins[s]

    @plsc.parallel_loop(0, _C, step=16, unroll=4)
    def _idx_body(o):
      ei = (iv[pl.ds(o, 16)] - 1) * 2 + viv[pl.ds(o, 16)]
      ej = (jv[pl.ds(o, 16)] - 1) * 2 + vjv[pl.ds(o, 16)]
      ei = jnp.where(ei < 0, ei + _NUM_NODES, ei)
      ej = jnp.where(ej < 0, ej + _NUM_NODES, ej)
      idx_xi[pl.ds(o, 16)] = ei
      idx_yi[pl.ds(o, 16)] = ei + _NUM_NODES
      idx_xj[pl.ds(o, 16)] = ej
      idx_yj[pl.ds(o, 16)] = ej + _NUM_NODES

  def issue_gathers(s):
    for idx, dst in zip(idxs, xys[s]):
      pltpu.async_copy(pos_hbm.at[idx], dst, sem_g)

  def drain_gathers(s):
    for buf in xys[s]:
      pltpu.make_async_copy(pos_hbm.at[pl.ds(0, _C)], buf, sem_g).wait()

  def pair_compute(s):
    x_i, y_i, x_j, y_j = xys[s]
    disv = ins[s][4]

    @plsc.parallel_loop(0, _C, step=16, unroll=8,
                        carry=jnp.zeros((16,), jnp.float32))
    def acc(o, a):
      dd = disv[pl.ds(o, 16)]
      dx = x_i[pl.ds(o, 16)] - x_j[pl.ds(o, 16)]
      dy = y_i[pl.ds(o, 16)] - y_j[pl.ds(o, 16)]
      d2 = jnp.maximum(dx * dx + dy * dy, 1e-30)
      bits = lax.bitcast_convert_type(d2, jnp.int32)
      r = lax.bitcast_convert_type(
          0x5F3759DF - lax.shift_right_arithmetic(bits, 1), jnp.float32)
      r = r * (1.5 - 0.5 * d2 * r * r)
      r = r * (1.5 - 0.5 * d2 * r * r)
      mag = d2 * r
      coeff = 0.25 / jnp.maximum(dd, lrvec)
      e = mag - dd
      return a + coeff * e * e

    accv[...] = accv[...] + acc

  issue_inputs(0, 0)
  for c in range(_CHUNKS):
    s = c % 2
    drain_inputs(s)
    idx_compute(s)
    issue_gathers(s)
    if c > 0:
      pair_compute(1 - s)
    if c + 1 < _CHUNKS:
      issue_inputs(c + 1, 1 - s)
    drain_gathers(s)
  pair_compute((_CHUNKS - 1) % 2)
  pltpu.sync_copy(accv, out_hbm.at[wid])


_mesh = plsc.VectorSubcoreMesh(core_axis_name="c", subcore_axis_name="s")
_scratch = (
    [pltpu.VMEM((_C,), jnp.int32)] * 4 + [pltpu.VMEM((_C,), jnp.float32)]
) * 2 + [
    pltpu.VMEM((_C,), jnp.int32)
] * 4 + [
    pltpu.VMEM((_C,), jnp.float32)
] * 8 + [
    pltpu.VMEM((16,), jnp.float32),
    pltpu.VMEM((16,), jnp.float32),
    pltpu.SemaphoreType.DMA,
    pltpu.SemaphoreType.DMA,
]
_call = pl.kernel(
    _stress_body,
    mesh=_mesh,
    out_type=jax.ShapeDtypeStruct((_NW, 16), jnp.float32),
    scratch_types=_scratch,
)


def kernel(i, j, vis_p_i, vis_p_j, dis, it, pos):
  lr = jnp.asarray(_LR_SCHEDULE, jnp.float32)[it]
  lr_vec = jnp.full((16,), lr, jnp.float32)
  out = _call(i.astype(jnp.int32), j.astype(jnp.int32),
              vis_p_i.astype(jnp.int32), vis_p_j.astype(jnp.int32),
              dis, lr_vec, pos.T.reshape(-1))
  return jnp.sum(out)


# speedup vs baseline: 1.0931x; 1.0931x over previous
import jax
import jax.numpy as jnp
from jax import lax
from jax.experimental import pallas as pl
from jax.experimental.pallas import tpu as pltpu
from jax.experimental.pallas import tpu_sc as plsc

_NUM_NODES = 2000000
_LR_SCHEDULE = (0.1, 0.095, 0.09, 0.085, 0.08, 0.075, 0.07, 0.065, 0.06, 0.055)
_B = 1048576
_NC = 2
_NS = 16
_NW = _NC * _NS
_C = 4096
_N_W = _B // _NW
_CHUNKS = _N_W // _C


def _stress_body(i_hbm, j_hbm, vi_hbm, vj_hbm, dis_hbm, lr_hbm, pos_hbm,
                 out_hbm,
                 iv0, jv0, viv0, vjv0, disv0,
                 iv1, jv1, viv1, vjv1, disv1,
                 idx_i, idx_j,
                 pi0, pj0, pi1, pj1,
                 lrv, accv, sem_in, sem_g):
  wid = lax.axis_index("s") * _NC + lax.axis_index("c")
  ins = ((iv0, jv0, viv0, vjv0, disv0), (iv1, jv1, viv1, vjv1, disv1))
  gbufs = ((pi0, pj0), (pi1, pj1))
  pltpu.sync_copy(lr_hbm, lrv)
  accv[...] = jnp.zeros((16,), jnp.float32)
  lrvec = lrv[...]

  def issue_inputs(c, s):
    base = wid * _N_W + c * _C
    for src, dst in zip((i_hbm, j_hbm, vi_hbm, vj_hbm, dis_hbm), ins[s]):
      pltpu.async_copy(src.at[pl.ds(base, _C)], dst, sem_in)

  def drain_inputs(s):
    for src, dst in zip((i_hbm, j_hbm, vi_hbm, vj_hbm, dis_hbm), ins[s]):
      pltpu.make_async_copy(src.at[pl.ds(0, _C)], dst, sem_in).wait()

  def idx_compute(s):
    iv, jv, viv, vjv, _ = ins[s]

    @plsc.parallel_loop(0, _C, step=16, unroll=4)
    def _idx_body(o):
      ei = (iv[pl.ds(o, 16)] - 1) * 2 + viv[pl.ds(o, 16)]
      ej = (jv[pl.ds(o, 16)] - 1) * 2 + vjv[pl.ds(o, 16)]
      idx_i[pl.ds(o, 16)] = jnp.where(ei < 0, ei + _NUM_NODES, ei)
      idx_j[pl.ds(o, 16)] = jnp.where(ej < 0, ej + _NUM_NODES, ej)

  def issue_gathers(s):
    p_i, p_j = gbufs[s]
    pltpu.async_copy(pos_hbm.at[idx_i], p_i, sem_g)
    pltpu.async_copy(pos_hbm.at[idx_j], p_j, sem_g)

  def drain_gathers(s):
    for buf in gbufs[s]:
      pltpu.make_async_copy(pos_hbm.at[pl.ds(0, _C)], buf, sem_g).wait()

  def pair_compute(s):
    p_i, p_j = gbufs[s]
    disv = ins[s][4]
    hi_mask = jnp.full((16,), -65536, jnp.int32)

    @plsc.parallel_loop(0, _C, step=16, unroll=8,
                        carry=jnp.zeros((16,), jnp.float32))
    def acc(o, a):
      dd = disv[pl.ds(o, 16)]
      wi = p_i[pl.ds(o, 16)]
      wj = p_j[pl.ds(o, 16)]
      x_i = lax.bitcast_convert_type(lax.shift_left(wi, 16), jnp.float32)
      y_i = lax.bitcast_convert_type(wi & hi_mask, jnp.float32)
      x_j = lax.bitcast_convert_type(lax.shift_left(wj, 16), jnp.float32)
      y_j = lax.bitcast_convert_type(wj & hi_mask, jnp.float32)
      dx = x_i - x_j
      dy = y_i - y_j
      d2 = jnp.maximum(dx * dx + dy * dy, 1e-30)
      bits = lax.bitcast_convert_type(d2, jnp.int32)
      r = lax.bitcast_convert_type(
          0x5F3759DF - lax.shift_right_arithmetic(bits, 1), jnp.float32)
      r = r * (1.5 - 0.5 * d2 * r * r)
      r = r * (1.5 - 0.5 * d2 * r * r)
      mag = d2 * r
      coeff = 0.25 / jnp.maximum(dd, lrvec)
      e = mag - dd
      return a + coeff * e * e

    accv[...] = accv[...] + acc

  issue_inputs(0, 0)
  for c in range(_CHUNKS):
    s = c % 2
    drain_inputs(s)
    idx_compute(s)
    issue_gathers(s)
    if c > 0:
      pair_compute(1 - s)
    if c + 1 < _CHUNKS:
      issue_inputs(c + 1, 1 - s)
    drain_gathers(s)
  pair_compute((_CHUNKS - 1) % 2)
  pltpu.sync_copy(accv, out_hbm.at[wid])


_mesh = plsc.VectorSubcoreMesh(core_axis_name="c", subcore_axis_name="s")
_scratch = (
    [pltpu.VMEM((_C,), jnp.int32)] * 4 + [pltpu.VMEM((_C,), jnp.float32)]
) * 2 + [
    pltpu.VMEM((_C,), jnp.int32)
] * 2 + [
    pltpu.VMEM((_C,), jnp.int32)
] * 4 + [
    pltpu.VMEM((16,), jnp.float32),
    pltpu.VMEM((16,), jnp.float32),
    pltpu.SemaphoreType.DMA,
    pltpu.SemaphoreType.DMA,
]
_call = pl.kernel(
    _stress_body,
    mesh=_mesh,
    out_type=jax.ShapeDtypeStruct((_NW, 16), jnp.float32),
    scratch_types=_scratch,
)


def kernel(i, j, vis_p_i, vis_p_j, dis, it, pos):
  lr = jnp.asarray(_LR_SCHEDULE, jnp.float32)[it]
  lr_vec = jnp.full((16,), lr, jnp.float32)
  pos_packed = lax.bitcast_convert_type(
      pos.astype(jnp.bfloat16), jnp.int32)
  out = _call(i.astype(jnp.int32), j.astype(jnp.int32),
              vis_p_i.astype(jnp.int32), vis_p_j.astype(jnp.int32),
              dis, lr_vec, pos_packed)
  return jnp.sum(out)


# speedup vs baseline: 1.2927x; 1.1826x over previous
import jax
import jax.numpy as jnp
from jax import lax
from jax.experimental import pallas as pl
from jax.experimental.pallas import tpu as pltpu
from jax.experimental.pallas import tpu_sc as plsc

_NUM_NODES = 2000000
_LR_SCHEDULE = (0.1, 0.095, 0.09, 0.085, 0.08, 0.075, 0.07, 0.065, 0.06, 0.055)
_B = 1048576
_NC = 2
_NS = 16
_NW = _NC * _NS
_C = 4096
_N_W = _B // _NW
_CHUNKS = _N_W // _C
_PC = 8000
_NPC = _NUM_NODES // _PC
_P1 = (_NPC + _NS - 1) // _NS


def _stress_body(i_hbm, j_hbm, vi_hbm, vj_hbm, dis_hbm, lr_hbm, pos_hbm,
                 out_hbm, scr0, scr1,
                 iv0, jv0, viv0, vjv0, disv0,
                 iv1, jv1, viv1, vjv1, disv1,
                 idx_i, idx_j,
                 pi0, pj0, pi1, pj1,
                 xb0, yb0, pk0, xb1, yb1, pk1,
                 lrv, accv, sem_in, sem_g, sem_p, sem_o):
  sid = lax.axis_index("s")
  nc = lax.axis_index("c")
  wid = sid * _NC + nc
  ins = ((iv0, jv0, viv0, vjv0, disv0), (iv1, jv1, viv1, vjv1, disv1))
  gbufs = ((pi0, pj0), (pi1, pj1))
  xbs, ybs, pks = (xb0, xb1), (yb0, yb1), (pk0, pk1)
  pltpu.sync_copy(lr_hbm, lrv)
  accv[...] = jnp.zeros((16,), jnp.float32)
  lrvec = lrv[...]

  def p1_issue(cc):
    c = cc * _NS + sid

    @pl.when(c < _NPC)
    def _():
      o = c * _PC
      pltpu.async_copy(pos_hbm.at[pl.ds(o, _PC)], xbs[cc % 2], sem_p)
      pltpu.async_copy(pos_hbm.at[pl.ds(_NUM_NODES + o, _PC)],
                       ybs[cc % 2], sem_p)

  def p1_drain_in(cc):
    c = cc * _NS + sid

    @pl.when(c < _NPC)
    def _():
      for buf in (xbs[cc % 2], ybs[cc % 2]):
        pltpu.make_async_copy(pos_hbm.at[pl.ds(0, _PC)], buf, sem_p).wait()

  def p1_pack(cc):
    c = cc * _NS + sid

    @pl.when(c < _NPC)
    def _():
      xb, yb, pk = xbs[cc % 2], ybs[cc % 2], pks[cc % 2]

      @plsc.parallel_loop(0, _PC, step=16, unroll=4)
      def _pack_body(o):
        xw = lax.bitcast_convert_type(xb[pl.ds(o, 16)], jnp.int32)
        yw = lax.bitcast_convert_type(yb[pl.ds(o, 16)], jnp.int32)
        xr = lax.shift_right_logical(
            xw + 0x7FFF + (lax.shift_right_logical(xw, 16) & 1), 16)
        yr = lax.shift_right_logical(
            yw + 0x7FFF + (lax.shift_right_logical(yw, 16) & 1), 16)
        pk[pl.ds(o, 16)] = xr | lax.shift_left(yr, 16)

      @pl.when(nc == 0)
      def _():
        pltpu.async_copy(pk, scr0.at[pl.ds(c * _PC, _PC)], sem_o)

      @pl.when(nc == 1)
      def _():
        pltpu.async_copy(pk, scr1.at[pl.ds(c * _PC, _PC)], sem_o)

  def p1_drain_out(cc):
    c = cc * _NS + sid

    @pl.when(c < _NPC)
    def _():
      pltpu.make_async_copy(pks[cc % 2], scr0.at[pl.ds(0, _PC)], sem_o).wait()

  p1_issue(0)
  for cc in range(_P1):
    if cc + 1 < _P1:
      p1_issue(cc + 1)
    if cc >= 2:
      p1_drain_out(cc - 2)
    p1_drain_in(cc)
    p1_pack(cc)
  p1_drain_out(_P1 - 2)
  p1_drain_out(_P1 - 1)
  plsc.subcore_barrier()

  def issue_inputs(c, s):
    base = wid * _N_W + c * _C
    for src, dst in zip((i_hbm, j_hbm, vi_hbm, vj_hbm, dis_hbm), ins[s]):
      pltpu.async_copy(src.at[pl.ds(base, _C)], dst, sem_in)

  def drain_inputs(s):
    for src, dst in zip((i_hbm, j_hbm, vi_hbm, vj_hbm, dis_hbm), ins[s]):
      pltpu.make_async_copy(src.at[pl.ds(0, _C)], dst, sem_in).wait()

  def idx_compute(s):
    iv, jv, viv, vjv, _ = ins[s]

    @plsc.parallel_loop(0, _C, step=16, unroll=4)
    def _idx_body(o):
      ei = (iv[pl.ds(o, 16)] - 1) * 2 + viv[pl.ds(o, 16)]
      ej = (jv[pl.ds(o, 16)] - 1) * 2 + vjv[pl.ds(o, 16)]
      idx_i[pl.ds(o, 16)] = jnp.where(ei < 0, ei + _NUM_NODES, ei)
      idx_j[pl.ds(o, 16)] = jnp.where(ej < 0, ej + _NUM_NODES, ej)

  def issue_gathers(s):
    p_i, p_j = gbufs[s]

    @pl.when(nc == 0)
    def _():
      pltpu.async_copy(scr0.at[idx_i], p_i, sem_g)
      pltpu.async_copy(scr0.at[idx_j], p_j, sem_g)

    @pl.when(nc == 1)
    def _():
      pltpu.async_copy(scr1.at[idx_i], p_i, sem_g)
      pltpu.async_copy(scr1.at[idx_j], p_j, sem_g)

  def drain_gathers(s):
    for buf in gbufs[s]:
      pltpu.make_async_copy(i_hbm.at[pl.ds(0, _C)], buf, sem_g).wait()

  def pair_compute(s):
    p_i, p_j = gbufs[s]
    disv = ins[s][4]
    hi_mask = jnp.full((16,), -65536, jnp.int32)

    @plsc.parallel_loop(0, _C, step=16, unroll=8,
                        carry=jnp.zeros((16,), jnp.float32))
    def acc(o, a):
      dd = disv[pl.ds(o, 16)]
      wi = p_i[pl.ds(o, 16)]
      wj = p_j[pl.ds(o, 16)]
      x_i = lax.bitcast_convert_type(lax.shift_left(wi, 16), jnp.float32)
      y_i = lax.bitcast_convert_type(wi & hi_mask, jnp.float32)
      x_j = lax.bitcast_convert_type(lax.shift_left(wj, 16), jnp.float32)
      y_j = lax.bitcast_convert_type(wj & hi_mask, jnp.float32)
      dx = x_i - x_j
      dy = y_i - y_j
      d2 = jnp.maximum(dx * dx + dy * dy, 1e-30)
      bits = lax.bitcast_convert_type(d2, jnp.int32)
      r = lax.bitcast_convert_type(
          0x5F3759DF - lax.shift_right_arithmetic(bits, 1), jnp.float32)
      r = r * (1.5 - 0.5 * d2 * r * r)
      r = r * (1.5 - 0.5 * d2 * r * r)
      mag = d2 * r
      coeff = 0.25 / jnp.maximum(dd, lrvec)
      e = mag - dd
      return a + coeff * e * e

    accv[...] = accv[...] + acc

  issue_inputs(0, 0)
  for c in range(_CHUNKS):
    s = c % 2
    drain_inputs(s)
    idx_compute(s)
    issue_gathers(s)
    if c > 0:
      pair_compute(1 - s)
    if c + 1 < _CHUNKS:
      issue_inputs(c + 1, 1 - s)
    drain_gathers(s)
  pair_compute((_CHUNKS - 1) % 2)
  pltpu.sync_copy(accv, out_hbm.at[wid])


_mesh = plsc.VectorSubcoreMesh(core_axis_name="c", subcore_axis_name="s")
_scratch = (
    [pltpu.VMEM((_C,), jnp.int32)] * 4 + [pltpu.VMEM((_C,), jnp.float32)]
) * 2 + [
    pltpu.VMEM((_C,), jnp.int32)
] * 2 + [
    pltpu.VMEM((_C,), jnp.int32)
] * 4 + [
    pltpu.VMEM((_PC,), jnp.float32),
    pltpu.VMEM((_PC,), jnp.float32),
    pltpu.VMEM((_PC,), jnp.int32),
    pltpu.VMEM((_PC,), jnp.float32),
    pltpu.VMEM((_PC,), jnp.float32),
    pltpu.VMEM((_PC,), jnp.int32),
    pltpu.VMEM((16,), jnp.float32),
    pltpu.VMEM((16,), jnp.float32),
    pltpu.SemaphoreType.DMA,
    pltpu.SemaphoreType.DMA,
    pltpu.SemaphoreType.DMA,
    pltpu.SemaphoreType.DMA,
]
_call = pl.kernel(
    _stress_body,
    mesh=_mesh,
    out_type=(jax.ShapeDtypeStruct((_NW, 16), jnp.float32),
              jax.ShapeDtypeStruct((_NUM_NODES,), jnp.int32),
              jax.ShapeDtypeStruct((_NUM_NODES,), jnp.int32)),
    scratch_types=_scratch,
)


def kernel(i, j, vis_p_i, vis_p_j, dis, it, pos):
  lr = jnp.asarray(_LR_SCHEDULE, jnp.float32)[it]
  lr_vec = jnp.full((16,), lr, jnp.float32)
  out, _, _ = _call(i.astype(jnp.int32), j.astype(jnp.int32),
                    vis_p_i.astype(jnp.int32), vis_p_j.astype(jnp.int32),
                    dis, lr_vec, pos.T.reshape(-1))
  return jnp.sum(out)


# speedup vs baseline: 1.3143x; 1.0167x over previous
import jax
import jax.numpy as jnp
from jax import lax
from jax.experimental import pallas as pl
from jax.experimental.pallas import tpu as pltpu
from jax.experimental.pallas import tpu_sc as plsc

_NUM_NODES = 2000000
_LR_SCHEDULE = (0.1, 0.095, 0.09, 0.085, 0.08, 0.075, 0.07, 0.065, 0.06, 0.055)
_B = 1048576
_NC = 2
_NS = 16
_NW = _NC * _NS
_C = 4096
_N_W = _B // _NW
_CHUNKS = _N_W // _C
_PC = 8000
_NPC = _NUM_NODES // _PC
_P1 = (_NPC + _NS - 1) // _NS


def _stress_body(i_hbm, j_hbm, vi_hbm, vj_hbm, dis_hbm, lr_hbm, pos_hbm,
                 out_hbm, scr0, scr1,
                 iv0, jv0, viv0, vjv0, disv0,
                 iv1, jv1, viv1, vjv1, disv1,
                 idx_i, idx_j,
                 pi0, pj0, pi1, pj1,
                 xb0, yb0, pk0, xb1, yb1, pk1,
                 lrv, accv, sem_in, sem_g, sem_p, sem_o):
  sid = lax.axis_index("s")
  nc = lax.axis_index("c")
  wid = sid * _NC + nc
  ins = ((iv0, jv0, viv0, vjv0, disv0), (iv1, jv1, viv1, vjv1, disv1))
  gbufs = ((pi0, pj0), (pi1, pj1))
  xbs, ybs, pks = (xb0, xb1), (yb0, yb1), (pk0, pk1)
  pltpu.sync_copy(lr_hbm, lrv)
  accv[...] = jnp.zeros((16,), jnp.float32)
  lrvec = lrv[...]

  def p1_issue(cc):
    c = cc * _NS + sid

    @pl.when(c < _NPC)
    def _():
      o = c * _PC
      pltpu.async_copy(pos_hbm.at[pl.ds(o, _PC)], xbs[cc % 2], sem_p)
      pltpu.async_copy(pos_hbm.at[pl.ds(_NUM_NODES + o, _PC)],
                       ybs[cc % 2], sem_p)

  def p1_drain_in(cc):
    c = cc * _NS + sid

    @pl.when(c < _NPC)
    def _():
      for buf in (xbs[cc % 2], ybs[cc % 2]):
        pltpu.make_async_copy(pos_hbm.at[pl.ds(0, _PC)], buf, sem_p).wait()

  def p1_pack(cc):
    c = cc * _NS + sid

    @pl.when(c < _NPC)
    def _():
      xb, yb, pk = xbs[cc % 2], ybs[cc % 2], pks[cc % 2]

      @plsc.parallel_loop(0, _PC, step=16, unroll=8)
      def _pack_body(o):
        xw = lax.bitcast_convert_type(xb[pl.ds(o, 16)], jnp.int32)
        yw = lax.bitcast_convert_type(yb[pl.ds(o, 16)], jnp.int32)
        xr = lax.shift_right_logical(xw + 0x8000, 16)
        yr = lax.shift_right_logical(yw + 0x8000, 16)
        pk[pl.ds(o, 16)] = xr | lax.shift_left(yr, 16)

      @pl.when(nc == 0)
      def _():
        pltpu.async_copy(pk, scr0.at[pl.ds(c * _PC, _PC)], sem_o)

      @pl.when(nc == 1)
      def _():
        pltpu.async_copy(pk, scr1.at[pl.ds(c * _PC, _PC)], sem_o)

  def p1_drain_out(cc):
    c = cc * _NS + sid

    @pl.when(c < _NPC)
    def _():
      pltpu.make_async_copy(pks[cc % 2], scr0.at[pl.ds(0, _PC)], sem_o).wait()

  p1_issue(0)
  for cc in range(_P1):
    if cc + 1 < _P1:
      p1_issue(cc + 1)
    if cc >= 2:
      p1_drain_out(cc - 2)
    p1_drain_in(cc)
    p1_pack(cc)
  p1_drain_out(_P1 - 2)
  p1_drain_out(_P1 - 1)
  plsc.subcore_barrier()

  def issue_inputs(c, s):
    base = wid * _N_W + c * _C
    for src, dst in zip((i_hbm, j_hbm, vi_hbm, vj_hbm, dis_hbm), ins[s]):
      pltpu.async_copy(src.at[pl.ds(base, _C)], dst, sem_in)

  def drain_inputs(s):
    for src, dst in zip((i_hbm, j_hbm, vi_hbm, vj_hbm, dis_hbm), ins[s]):
      pltpu.make_async_copy(src.at[pl.ds(0, _C)], dst, sem_in).wait()

  def idx_compute(s):
    iv, jv, viv, vjv, _ = ins[s]

    @plsc.parallel_loop(0, _C, step=16, unroll=4)
    def _idx_body(o):
      ei = (iv[pl.ds(o, 16)] - 1) * 2 + viv[pl.ds(o, 16)]
      ej = (jv[pl.ds(o, 16)] - 1) * 2 + vjv[pl.ds(o, 16)]
      idx_i[pl.ds(o, 16)] = jnp.where(ei < 0, ei + _NUM_NODES, ei)
      idx_j[pl.ds(o, 16)] = jnp.where(ej < 0, ej + _NUM_NODES, ej)

  def issue_gathers(s):
    p_i, p_j = gbufs[s]

    @pl.when(nc == 0)
    def _():
      pltpu.async_copy(scr0.at[idx_i], p_i, sem_g)
      pltpu.async_copy(scr0.at[idx_j], p_j, sem_g)

    @pl.when(nc == 1)
    def _():
      pltpu.async_copy(scr1.at[idx_i], p_i, sem_g)
      pltpu.async_copy(scr1.at[idx_j], p_j, sem_g)

  def drain_gathers(s):
    for buf in gbufs[s]:
      pltpu.make_async_copy(i_hbm.at[pl.ds(0, _C)], buf, sem_g).wait()

  def pair_compute(s):
    p_i, p_j = gbufs[s]
    disv = ins[s][4]
    hi_mask = jnp.full((16,), -65536, jnp.int32)

    @plsc.parallel_loop(0, _C, step=16, unroll=8,
                        carry=jnp.zeros((16,), jnp.float32))
    def acc(o, a):
      dd = disv[pl.ds(o, 16)]
      wi = p_i[pl.ds(o, 16)]
      wj = p_j[pl.ds(o, 16)]
      x_i = lax.bitcast_convert_type(lax.shift_left(wi, 16), jnp.float32)
      y_i = lax.bitcast_convert_type(wi & hi_mask, jnp.float32)
      x_j = lax.bitcast_convert_type(lax.shift_left(wj, 16), jnp.float32)
      y_j = lax.bitcast_convert_type(wj & hi_mask, jnp.float32)
      dx = x_i - x_j
      dy = y_i - y_j
      d2 = jnp.maximum(dx * dx + dy * dy, 1e-30)
      bits = lax.bitcast_convert_type(d2, jnp.int32)
      r = lax.bitcast_convert_type(
          0x5F3759DF - lax.shift_right_arithmetic(bits, 1), jnp.float32)
      r = r * (1.5 - 0.5 * d2 * r * r)
      r = r * (1.5 - 0.5 * d2 * r * r)
      mag = d2 * r
      coeff = 0.25 / jnp.maximum(dd, lrvec)
      e = mag - dd
      return a + coeff * e * e

    accv[...] = accv[...] + acc

  issue_inputs(0, 0)
  for c in range(_CHUNKS):
    s = c % 2
    drain_inputs(s)
    idx_compute(s)
    issue_gathers(s)
    if c > 0:
      pair_compute(1 - s)
    if c + 1 < _CHUNKS:
      issue_inputs(c + 1, 1 - s)
    drain_gathers(s)
  pair_compute((_CHUNKS - 1) % 2)
  pltpu.sync_copy(accv, out_hbm.at[wid])


_mesh = plsc.VectorSubcoreMesh(core_axis_name="c", subcore_axis_name="s")
_scratch = (
    [pltpu.VMEM((_C,), jnp.int32)] * 4 + [pltpu.VMEM((_C,), jnp.float32)]
) * 2 + [
    pltpu.VMEM((_C,), jnp.int32)
] * 2 + [
    pltpu.VMEM((_C,), jnp.int32)
] * 4 + [
    pltpu.VMEM((_PC,), jnp.float32),
    pltpu.VMEM((_PC,), jnp.float32),
    pltpu.VMEM((_PC,), jnp.int32),
    pltpu.VMEM((_PC,), jnp.float32),
    pltpu.VMEM((_PC,), jnp.float32),
    pltpu.VMEM((_PC,), jnp.int32),
    pltpu.VMEM((16,), jnp.float32),
    pltpu.VMEM((16,), jnp.float32),
    pltpu.SemaphoreType.DMA,
    pltpu.SemaphoreType.DMA,
    pltpu.SemaphoreType.DMA,
    pltpu.SemaphoreType.DMA,
]
_call = pl.kernel(
    _stress_body,
    mesh=_mesh,
    out_type=(jax.ShapeDtypeStruct((_NW, 16), jnp.float32),
              jax.ShapeDtypeStruct((_NUM_NODES,), jnp.int32),
              jax.ShapeDtypeStruct((_NUM_NODES,), jnp.int32)),
    scratch_types=_scratch,
)


def kernel(i, j, vis_p_i, vis_p_j, dis, it, pos):
  lr = jnp.asarray(_LR_SCHEDULE, jnp.float32)[it]
  lr_vec = jnp.full((16,), lr, jnp.float32)
  out, _, _ = _call(i.astype(jnp.int32), j.astype(jnp.int32),
                    vis_p_i.astype(jnp.int32), vis_p_j.astype(jnp.int32),
                    dis, lr_vec, pos.T.reshape(-1))
  return jnp.sum(out)
